# baseline (speedup 1.0000x reference)
import functools

import jax
import jax.numpy as jnp
from jax import lax
from jax.experimental import pallas as pl
from jax.experimental.pallas import tpu as pltpu
from jax.experimental.pallas import tpu_sc as plsc

N = 10000
E = 160000
D = 256
HEADS = 4
HEAD_DIM = 64
HALF = 128
HID4 = 64

NC = 2
NS = 16
EPT = E // NS
CH = 80
NCHUNK = EPT // CH
RPT = N // NS

_SC_MESH = dict(core_axis_name="c", subcore_axis_name="s")


def _lane_bcast(v, i):
    idx = jnp.full((16, 1), i, jnp.int32)
    dn = lax.GatherDimensionNumbers(offset_dims=(), collapsed_slice_dims=(0,),
                                    start_index_map=(0,))
    return lax.gather(v, idx, dn, (1,),
                      mode=lax.GatherScatterMode.PROMISE_IN_BOUNDS)


def _seg_sum_body(with_counts, with_gat, refs):
    if with_gat:
        (hlo, hhi, srcv, dstv, asv, adv, zlo, zcnt,
         aglo, aghi, den_out,
         sidx0, sidx1, sidx2, didx0, didx1, didx2,
         rows0, rows1, rows2, asb, adb, eebuf, acc, cntacc,
         gsem0, gsem1, gsem2, ssem0, ssem1, ssem2,
         isem0, isem1, isem2) = refs
    elif with_counts:
        (hlo, hhi, srcv, dstv, zlo, zcnt, ones_h,
         aglo, aghi, cnt_out,
         sidx0, sidx1, sidx2, didx0, didx1, didx2,
         rows0, rows1, rows2, ones, acc, cntacc,
         gsem0, gsem1, gsem2, ssem0, ssem1, ssem2,
         isem0, isem1, isem2) = refs
    else:
        (hlo, hhi, srcv, dstv, zlo,
         aglo, aghi,
         sidx0, sidx1, sidx2, didx0, didx1, didx2,
         rows0, rows1, rows2, acc,
         gsem0, gsem1, gsem2, ssem0, ssem1, ssem2,
         isem0, isem1, isem2) = refs
    sidx = (sidx0, sidx1, sidx2)
    didx = (didx0, didx1, didx2)
    rows = (rows0, rows1, rows2)
    gsem = (gsem0, gsem1, gsem2)
    ssem = (ssem0, ssem1, ssem2)
    isem = (isem0, isem1, isem2)

    cid = lax.axis_index("c")
    sid = lax.axis_index("s")

    pltpu.sync_copy(zlo.at[sid], acc.at[pl.ds(sid * RPT, RPT)])
    if with_counts or with_gat:
        pltpu.sync_copy(zcnt.at[sid], cntacc.at[pl.ds(sid * RPT, RPT)])
    if with_counts:
        pltpu.sync_copy(ones_h, ones)

    def idx_copies(k, b):
        base = sid * EPT + k * CH
        return ((srcv.at[pl.ds(base, CH)], sidx[b]),
                (dstv.at[pl.ds(base, CH)], didx[b]))

    def load_idx_sync(k, b):
        for s, d in idx_copies(k, b):
            pltpu.sync_copy(s, d)

    def start_idx(k, b):
        for s, d in idx_copies(k, b):
            pltpu.async_copy(s, d, isem[b])

    def wait_idx(k, b):
        for s, d in idx_copies(k, b):
            pltpu.make_async_copy(s, d, isem[b]).wait()

    def gather_copies(b):
        copies = [(hlo.at[sidx[b]], rows[b]), (hhi.at[sidx[b]], rows[b])]
        if with_gat:
            copies.append((asv.at[sidx[b]], asb))
            copies.append((adv.at[didx[b]], adb))
        return copies

    def start_gather(b):
        cps = gather_copies(b)

        @pl.when(cid == 0)
        def _():
            pltpu.async_copy(cps[0][0], cps[0][1], gsem[b])

        @pl.when(cid == 1)
        def _():
            pltpu.async_copy(cps[1][0], cps[1][1], gsem[b])

        for s, d in cps[2:]:
            pltpu.async_copy(s, d, gsem[b])

    def wait_gather(b):
        cps = gather_copies(b)
        pltpu.make_async_copy(cps[0][0], cps[0][1], gsem[b]).wait()
        for s, d in cps[2:]:
            pltpu.make_async_copy(s, d, gsem[b]).wait()

    def compute_ee(b):
        @plsc.parallel_loop(0, CH, 1, unroll=4)
        def _(e):
            ev = asb[e] + adb[e]
            ev = jnp.maximum(ev, 0.2 * ev)
            eebuf[e] = jnp.exp(ev)

    def compute_and_scatter(j, b):
        if with_gat:
            h0 = cid * 2
            rb = rows[b]

            @plsc.parallel_loop(0, CH, 1, unroll=4)
            def _(e):
                ee = eebuf[e]
                w0 = _lane_bcast(ee, h0)
                w1 = _lane_bcast(ee, h0 + 1)
                for j2 in range(8):
                    w = w0 if j2 < 4 else w1
                    rb[e, pl.ds(j2 * 16, 16)] = rb[e, pl.ds(j2 * 16, 16)] * w

            pltpu.async_copy(rb, acc.at[didx[b]], ssem[b], add=True)

            @pl.when(cid == j % 2)
            def _():
                pltpu.sync_copy(eebuf, cntacc.at[didx[b]], add=True)
        else:
            pltpu.async_copy(rows[b], acc.at[didx[b]], ssem[b], add=True)
            if with_counts:
                @pl.when(cid == j % 2)
                def _():
                    pltpu.sync_copy(ones, cntacc.at[didx[b]], add=True)

    def wait_scatter(b):
        pltpu.make_async_copy(rows[b], acc.at[didx[b]], ssem[b]).wait()

    def chunk(j, b, gather_next, prefetch_idx):
        bn = (b + 1) % 3
        bp = (b + 2) % 3
        wait_gather(b)
        if with_gat:
            compute_ee(b)
        if gather_next:
            @pl.when(j >= 1)
            def _():
                wait_idx(j + 1, bn)
            start_gather(bn)
        compute_and_scatter(j, b)

        @pl.when(j >= 1)
        def _():
            wait_scatter(bp)
        if prefetch_idx:
            start_idx(j + 2, bp)

    plsc.subcore_barrier()
    load_idx_sync(0, 0)
    load_idx_sync(1, 1)
    start_gather(0)

    def triple(j3, carry):
        chunk(j3 * 3, 0, True, True)
        chunk(j3 * 3 + 1, 1, True, True)
        chunk(j3 * 3 + 2, 2, True, True)
        return carry

    nfull = (NCHUNK - 2) // 3
    lax.fori_loop(0, nfull, triple, 0)
    chunk(NCHUNK - 2, 0, True, False)
    chunk(NCHUNK - 1, 1, False, False)
    wait_scatter((NCHUNK - 1) % 3)
    plsc.subcore_barrier()

    @pl.when(cid == 0)
    def _():
        pltpu.sync_copy(acc.at[pl.ds(sid * RPT, RPT)], aglo.at[sid])

    @pl.when(cid == 1)
    def _():
        pltpu.sync_copy(acc.at[pl.ds(sid * RPT, RPT)], aghi.at[sid])

    if with_counts or with_gat:
        small_out = cnt_out if with_counts else den_out
        pltpu.sync_copy(cntacc.at[pl.ds(sid * RPT, RPT)],
                        small_out.at[cid, sid])


def _sc_seg_sum(hlo, hhi, srcv, dstv, with_counts):
    f32 = jnp.float32
    out_type = [jax.ShapeDtypeStruct((NS, RPT, HALF), f32),
                jax.ShapeDtypeStruct((NS, RPT, HALF), f32)]
    scratch = [pltpu.VMEM((CH,), jnp.int32)] * 6 \
        + [pltpu.VMEM((CH, HALF), f32)] * 3
    args = [hlo, hhi, srcv, dstv, jnp.zeros((NS, RPT, HALF), f32)]
    if with_counts:
        out_type.append(jax.ShapeDtypeStruct((NC, NS, RPT, 16), f32))
        args += [jnp.zeros((NS, RPT, 16), f32), jnp.ones((CH, 16), f32)]
        scratch.append(pltpu.VMEM((CH, 16), f32))
        scratch.append(pltpu.VMEM_SHARED((N, HALF), f32))
        scratch.append(pltpu.VMEM_SHARED((N, 16), f32))
    else:
        scratch.append(pltpu.VMEM_SHARED((N, HALF), f32))
    scratch += [pltpu.SemaphoreType.DMA] * 9

    body = functools.partial(_seg_sum_body, with_counts, False)
    fn = pl.kernel(lambda *refs: body(refs), out_type=tuple(out_type),
                   mesh=plsc.VectorSubcoreMesh(**_SC_MESH),
                   scratch_types=tuple(scratch),
                   compiler_params=pltpu.CompilerParams(
                       use_tc_tiling_on_sc=False))
    return fn(*args)


def _sc_gat(xlo, xhi, asv, adv, srcv, dstv):
    f32 = jnp.float32
    out_type = (jax.ShapeDtypeStruct((NS, RPT, HALF), f32),
                jax.ShapeDtypeStruct((NS, RPT, HALF), f32),
                jax.ShapeDtypeStruct((NC, NS, RPT, 16), f32))
    scratch = ((pltpu.VMEM((CH,), jnp.int32),) * 6
               + (pltpu.VMEM((CH, HALF), f32),) * 3
               + (pltpu.VMEM((CH, 16), f32),) * 3
               + (pltpu.VMEM_SHARED((N, HALF), f32),
                  pltpu.VMEM_SHARED((N, 16), f32))
               + (pltpu.SemaphoreType.DMA,) * 9)
    body = functools.partial(_seg_sum_body, False, True)
    fn = pl.kernel(lambda *refs: body(refs), out_type=out_type,
                   mesh=plsc.VectorSubcoreMesh(**_SC_MESH),
                   scratch_types=scratch,
                   compiler_params=pltpu.CompilerParams(
                       use_tc_tiling_on_sc=False))
    return fn(xlo, xhi, srcv, dstv, asv, adv,
              jnp.zeros((NS, RPT, HALF), f32), jnp.zeros((NS, RPT, 16), f32))


def _mm_t(a, w):
    return lax.dot_general(a, w, (((1,), (1,)), ((), ())),
                           preferred_element_type=jnp.float32)


def _bn_relu(z, g, b):
    m = jnp.mean(z, axis=0)
    v = jnp.mean((z - m) * (z - m), axis=0)
    return jnp.maximum((z - m) * lax.rsqrt(v + 1e-5) * g + b, 0.0)


def _tc_call(body, out_shape, *args):
    return pl.pallas_call(
        body, out_shape=out_shape,
        compiler_params=pltpu.CompilerParams(
            vmem_limit_bytes=128 * 1024 * 1024),
    )(*args)


def _tc_combine(aglo, aghi, cnt0, cnt1, hplo, hphi, Wl, bl, Wr, g, b):
    def body(aglo_r, aghi_r, c0_r, c1_r, hlo_r, hhi_r, wl_r, bl_r, wr_r,
             g_r, b_r, olo_r, ohi_r):
        agg = jnp.concatenate([aglo_r[...], aghi_r[...]], axis=1)
        cnt = jnp.maximum(c0_r[...][:, :1] + c1_r[...][:, :1], 1.0)
        h_prev = jnp.concatenate([hlo_r[...], hhi_r[...]], axis=1)
        z = (_mm_t(agg / cnt, wl_r[...]) + bl_r[...][None, :]
             + _mm_t(h_prev, wr_r[...]))
        res = _bn_relu(z, g_r[...][None, :], b_r[...][None, :])
        olo_r[...] = res[:, :HALF]
        ohi_r[...] = res[:, HALF:]

    f32 = jnp.float32
    out_shape = (jax.ShapeDtypeStruct((N, HALF), f32),
                 jax.ShapeDtypeStruct((N, HALF), f32))
    return _tc_call(body, out_shape, aglo, aghi, cnt0, cnt1, hplo, hphi,
                    Wl, bl, Wr, g, b)


def _tc_combine_gat_prep(aglo, aghi, cnt0, cnt1, hplo, hphi, Wl, bl, Wr, g, b,
                         Wg, As16, Ad16):
    def body(aglo_r, aghi_r, c0_r, c1_r, hlo_r, hhi_r, wl_r, bl_r, wr_r,
             g_r, b_r, wg_r, as_r, ad_r, xlo_r, xhi_r, aso_r, ado_r):
        agg = jnp.concatenate([aglo_r[...], aghi_r[...]], axis=1)
        cnt = jnp.maximum(c0_r[...][:, :1] + c1_r[...][:, :1], 1.0)
        h_prev = jnp.concatenate([hlo_r[...], hhi_r[...]], axis=1)
        z = (_mm_t(agg / cnt, wl_r[...]) + bl_r[...][None, :]
             + _mm_t(h_prev, wr_r[...]))
        h = _bn_relu(z, g_r[...][None, :], b_r[...][None, :])
        xl = _mm_t(h, wg_r[...])
        xlo_r[...] = xl[:, :HALF]
        xhi_r[...] = xl[:, HALF:]
        aso_r[...] = jnp.dot(xl, as_r[...],
                             preferred_element_type=jnp.float32)
        ado_r[...] = jnp.dot(xl, ad_r[...],
                             preferred_element_type=jnp.float32)

    f32 = jnp.float32
    out_shape = (jax.ShapeDtypeStruct((N, HALF), f32),
                 jax.ShapeDtypeStruct((N, HALF), f32),
                 jax.ShapeDtypeStruct((N, 16), f32),
                 jax.ShapeDtypeStruct((N, 16), f32))
    return _tc_call(body, out_shape, aglo, aghi, cnt0, cnt1, hplo, hphi,
                    Wl, bl, Wr, g, b, Wg, As16, Ad16)


def _tc_final(wlo, whi, den0, den1, S16, bg, gg, bgb, Wc1, bc1, Wc2, bc2,
              Wc3, bc3):
    def body(wlo_r, whi_r, d0_r, d1_r, s_r, bg_r, gg_r, bgb_r, w1_r, b1_r,
             w2_r, b2_r, w3_r, b3_r, probs_r, emb_r):
        wacc = jnp.concatenate([wlo_r[...], whi_r[...]], axis=1)
        den_b = jnp.dot(d0_r[...] + d1_r[...], s_r[...],
                        preferred_element_type=jnp.float32)
        h = wacc / jnp.maximum(den_b, 1e-16) + bg_r[...][None, :]
        h = _bn_relu(h, gg_r[...][None, :], bgb_r[...][None, :])
        emb_r[...] = h
        c = jnp.maximum(_mm_t(h, w1_r[...]) + b1_r[...][None, :], 0.0)
        c = jnp.maximum(_mm_t(c, w2_r[...]) + b2_r[...][None, :], 0.0)
        logits = _mm_t(c, w3_r[...]) + b3_r[...][None, :]
        probs_r[...] = 1.0 / (1.0 + jnp.exp(-logits))

    f32 = jnp.float32
    out_shape = (jax.ShapeDtypeStruct((N, HALF), f32),
                 jax.ShapeDtypeStruct((N, D), f32))
    return _tc_call(body, out_shape, wlo, whi, den0, den1, S16, bg, gg, bgb,
                    Wc1, bc1, Wc2, bc2, Wc3, bc3)


def kernel(x, edge_index, Wl1, bl1, Wr1, g1, b1, Wl2, bl2, Wr2, g2, b2,
           Wg, att_src, att_dst, bg, gg, bgb, Wc1, bc1, Wc2, bc2, Wc3, bc3):
    f32 = jnp.float32
    src = edge_index[0]
    dst = edge_index[1]
    x_lo = x[:, :HALF]
    x_hi = x[:, HALF:]

    lane = jnp.arange(D)
    As16 = jnp.zeros((D, 16), f32).at[lane, lane // HEAD_DIM].set(
        att_src.reshape(D))
    Ad16 = jnp.zeros((D, 16), f32).at[lane, lane // HEAD_DIM].set(
        att_dst.reshape(D))
    S16 = jnp.zeros((16, D), f32).at[lane // HEAD_DIM, lane].set(1.0)
    Wc3p = jnp.zeros((HALF, HID4), f32).at[0].set(Wc3[0])
    bc3p = jnp.zeros((HALF,), f32).at[0].set(bc3[0])

    ag1_lo, ag1_hi, cnt2 = _sc_seg_sum(x_lo, x_hi, src, dst,
                                       with_counts=True)
    cnt2 = cnt2.reshape(NC, N, 16)
    cnt0, cnt1 = cnt2[0], cnt2[1]
    h1_lo, h1_hi = _tc_combine(ag1_lo.reshape(N, HALF),
                               ag1_hi.reshape(N, HALF), cnt0, cnt1,
                               x_lo, x_hi, Wl1, bl1, Wr1, g1, b1)

    ag2_lo, ag2_hi = _sc_seg_sum(h1_lo, h1_hi, src, dst, with_counts=False)
    xl_lo, xl_hi, as16, ad16 = _tc_combine_gat_prep(
        ag2_lo.reshape(N, HALF), ag2_hi.reshape(N, HALF), cnt0, cnt1,
        h1_lo, h1_hi, Wl2, bl2, Wr2, g2, b2, Wg, As16, Ad16)

    w_lo, w_hi, den2 = _sc_gat(xl_lo, xl_hi, as16, ad16, src, dst)
    den2 = den2.reshape(NC, N, 16)
    probs, emb = _tc_final(w_lo.reshape(N, HALF), w_hi.reshape(N, HALF),
                           den2[0], den2[1], S16, bg, gg, bgb,
                           Wc1, bc1, Wc2, bc2, Wc3p, bc3p)
    return probs[:, 0], emb


# speedup vs baseline: 1.0721x; 1.0721x over previous
import functools

import jax
import jax.numpy as jnp
from jax import lax
from jax.experimental import pallas as pl
from jax.experimental.pallas import tpu as pltpu
from jax.experimental.pallas import tpu_sc as plsc

N = 10000
E = 160000
D = 256
HEADS = 4
HEAD_DIM = 64
HALF = 128
HID4 = 64

NC = 2
NS = 16
EPT = E // NS
CH = 80
NCHUNK = EPT // CH
RPT = N // NS

_SC_MESH = dict(core_axis_name="c", subcore_axis_name="s")


def _lane_bcast(v, i):
    idx = jnp.full((16, 1), i, jnp.int32)
    dn = lax.GatherDimensionNumbers(offset_dims=(), collapsed_slice_dims=(0,),
                                    start_index_map=(0,))
    return lax.gather(v, idx, dn, (1,),
                      mode=lax.GatherScatterMode.PROMISE_IN_BOUNDS)


def _seg_sum_body(with_counts, with_gat, refs):
    if with_gat:
        (hlo, hhi, srcv, dstv, asv, adv, zlo, zcnt,
         aglo, aghi, den_out,
         sidx0, sidx1, sidx2, didx0, didx1, didx2,
         rows0, rows1, rows2, asb, adb, eebuf, acc, cntacc,
         gsem0, gsem1, gsem2, ssem0, ssem1, ssem2,
         isem0, isem1, isem2) = refs
    elif with_counts:
        (hlo, hhi, srcv, dstv, zlo, zcnt, ones_h,
         aglo, aghi, cnt_out,
         sidx0, sidx1, sidx2, didx0, didx1, didx2,
         rows0, rows1, rows2, ones, acc, cntacc,
         gsem0, gsem1, gsem2, ssem0, ssem1, ssem2,
         isem0, isem1, isem2) = refs
    else:
        (hlo, hhi, srcv, dstv, zlo,
         aglo, aghi,
         sidx0, sidx1, sidx2, didx0, didx1, didx2,
         rows0, rows1, rows2, acc,
         gsem0, gsem1, gsem2, ssem0, ssem1, ssem2,
         isem0, isem1, isem2) = refs
    sidx = (sidx0, sidx1, sidx2)
    didx = (didx0, didx1, didx2)
    rows = (rows0, rows1, rows2)
    gsem = (gsem0, gsem1, gsem2)
    ssem = (ssem0, ssem1, ssem2)
    isem = (isem0, isem1, isem2)

    cid = lax.axis_index("c")
    sid = lax.axis_index("s")

    pltpu.sync_copy(zlo.at[sid], acc.at[pl.ds(sid * RPT, RPT)])
    if with_counts or with_gat:
        pltpu.sync_copy(zcnt.at[sid], cntacc.at[pl.ds(sid * RPT, RPT)])
    if with_counts:
        pltpu.sync_copy(ones_h, ones)

    def idx_copies(k, b):
        base = sid * EPT + k * CH
        return ((srcv.at[pl.ds(base, CH)], sidx[b]),
                (dstv.at[pl.ds(base, CH)], didx[b]))

    def load_idx_sync(k, b):
        for s, d in idx_copies(k, b):
            pltpu.sync_copy(s, d)

    def start_idx(k, b):
        for s, d in idx_copies(k, b):
            pltpu.async_copy(s, d, isem[b])

    def wait_idx(k, b):
        for s, d in idx_copies(k, b):
            pltpu.make_async_copy(s, d, isem[b]).wait()

    def gather_copies(b):
        copies = [(hlo.at[sidx[b]], rows[b]), (hhi.at[sidx[b]], rows[b])]
        if with_gat:
            copies.append((asv.at[sidx[b]], asb))
            copies.append((adv.at[didx[b]], adb))
        return copies

    def start_gather(b):
        cps = gather_copies(b)

        @pl.when(cid == 0)
        def _():
            pltpu.async_copy(cps[0][0], cps[0][1], gsem[b])

        @pl.when(cid == 1)
        def _():
            pltpu.async_copy(cps[1][0], cps[1][1], gsem[b])

        for s, d in cps[2:]:
            pltpu.async_copy(s, d, gsem[b])

    def wait_gather(b):
        cps = gather_copies(b)
        pltpu.make_async_copy(cps[0][0], cps[0][1], gsem[b]).wait()
        for s, d in cps[2:]:
            pltpu.make_async_copy(s, d, gsem[b]).wait()

    def compute_ee(b):
        @plsc.parallel_loop(0, CH, 1, unroll=4)
        def _(e):
            ev = asb[e] + adb[e]
            ev = jnp.maximum(ev, 0.2 * ev)
            eebuf[e] = jnp.exp(ev)

    def compute_and_scatter(j, b):
        if with_gat:
            h0 = cid * 2
            rb = rows[b]

            @plsc.parallel_loop(0, CH, 1, unroll=4)
            def _(e):
                ee = eebuf[e]
                w0 = _lane_bcast(ee, h0)
                w1 = _lane_bcast(ee, h0 + 1)
                for j2 in range(8):
                    w = w0 if j2 < 4 else w1
                    rb[e, pl.ds(j2 * 16, 16)] = rb[e, pl.ds(j2 * 16, 16)] * w

            pltpu.async_copy(rb, acc.at[didx[b]], ssem[b], add=True)

            @pl.when(cid == 0)
            def _():
                pltpu.sync_copy(eebuf, cntacc.at[didx[b]], add=True)
        else:
            pltpu.async_copy(rows[b], acc.at[didx[b]], ssem[b], add=True)
            if with_counts:
                @pl.when(cid == 0)
                def _():
                    pltpu.sync_copy(ones, cntacc.at[didx[b]], add=True)

    def wait_scatter(b):
        pltpu.make_async_copy(rows[b], acc.at[didx[b]], ssem[b]).wait()

    def chunk(j, b, gather_next, prefetch_idx):
        bn = (b + 1) % 3
        bp = (b + 2) % 3
        wait_gather(b)
        if with_gat:
            compute_ee(b)
        if gather_next:
            @pl.when(j >= 1)
            def _():
                wait_idx(j + 1, bn)
            start_gather(bn)
        compute_and_scatter(j, b)

        @pl.when(j >= 1)
        def _():
            wait_scatter(bp)
        if prefetch_idx:
            start_idx(j + 2, bp)

    plsc.subcore_barrier()
    load_idx_sync(0, 0)
    load_idx_sync(1, 1)
    start_gather(0)

    def triple(j3, carry):
        chunk(j3 * 3, 0, True, True)
        chunk(j3 * 3 + 1, 1, True, True)
        chunk(j3 * 3 + 2, 2, True, True)
        return carry

    nfull = (NCHUNK - 2) // 3
    lax.fori_loop(0, nfull, triple, 0)
    chunk(NCHUNK - 2, 0, True, False)
    chunk(NCHUNK - 1, 1, False, False)
    wait_scatter((NCHUNK - 1) % 3)
    plsc.subcore_barrier()

    @pl.when(cid == 0)
    def _():
        pltpu.sync_copy(acc.at[pl.ds(sid * RPT, RPT)], aglo.at[sid])

    @pl.when(cid == 1)
    def _():
        pltpu.sync_copy(acc.at[pl.ds(sid * RPT, RPT)], aghi.at[sid])

    if with_counts or with_gat:
        small_out = cnt_out if with_counts else den_out

        @pl.when(cid == 0)
        def _():
            pltpu.sync_copy(cntacc.at[pl.ds(sid * RPT, RPT)],
                            small_out.at[sid])


def _sc_seg_sum(hlo, hhi, srcv, dstv, with_counts):
    f32 = jnp.float32
    out_type = [jax.ShapeDtypeStruct((NS, RPT, HALF), f32),
                jax.ShapeDtypeStruct((NS, RPT, HALF), f32)]
    scratch = [pltpu.VMEM((CH,), jnp.int32)] * 6 \
        + [pltpu.VMEM((CH, HALF), f32)] * 3
    args = [hlo, hhi, srcv, dstv, jnp.zeros((NS, RPT, HALF), f32)]
    if with_counts:
        out_type.append(jax.ShapeDtypeStruct((NS, RPT, 16), f32))
        args += [jnp.zeros((NS, RPT, 16), f32), jnp.ones((CH, 16), f32)]
        scratch.append(pltpu.VMEM((CH, 16), f32))
        scratch.append(pltpu.VMEM_SHARED((N, HALF), f32))
        scratch.append(pltpu.VMEM_SHARED((N, 16), f32))
    else:
        scratch.append(pltpu.VMEM_SHARED((N, HALF), f32))
    scratch += [pltpu.SemaphoreType.DMA] * 9

    body = functools.partial(_seg_sum_body, with_counts, False)
    fn = pl.kernel(lambda *refs: body(refs), out_type=tuple(out_type),
                   mesh=plsc.VectorSubcoreMesh(**_SC_MESH),
                   scratch_types=tuple(scratch),
                   compiler_params=pltpu.CompilerParams(
                       use_tc_tiling_on_sc=False))
    return fn(*args)


def _sc_gat(xlo, xhi, asv, adv, srcv, dstv):
    f32 = jnp.float32
    out_type = (jax.ShapeDtypeStruct((NS, RPT, HALF), f32),
                jax.ShapeDtypeStruct((NS, RPT, HALF), f32),
                jax.ShapeDtypeStruct((NS, RPT, 16), f32))
    scratch = ((pltpu.VMEM((CH,), jnp.int32),) * 6
               + (pltpu.VMEM((CH, HALF), f32),) * 3
               + (pltpu.VMEM((CH, 16), f32),) * 3
               + (pltpu.VMEM_SHARED((N, HALF), f32),
                  pltpu.VMEM_SHARED((N, 16), f32))
               + (pltpu.SemaphoreType.DMA,) * 9)
    body = functools.partial(_seg_sum_body, False, True)
    fn = pl.kernel(lambda *refs: body(refs), out_type=out_type,
                   mesh=plsc.VectorSubcoreMesh(**_SC_MESH),
                   scratch_types=scratch,
                   compiler_params=pltpu.CompilerParams(
                       use_tc_tiling_on_sc=False))
    return fn(xlo, xhi, srcv, dstv, asv, adv,
              jnp.zeros((NS, RPT, HALF), f32), jnp.zeros((NS, RPT, 16), f32))


def _mm_t(a, w):
    return lax.dot_general(a, w, (((1,), (1,)), ((), ())),
                           preferred_element_type=jnp.float32)


def _bn_relu(z, g, b):
    m = jnp.mean(z, axis=0)
    v = jnp.mean((z - m) * (z - m), axis=0)
    return jnp.maximum((z - m) * lax.rsqrt(v + 1e-5) * g + b, 0.0)


def _tc_call(body, out_shape, *args):
    return pl.pallas_call(
        body, out_shape=out_shape,
        compiler_params=pltpu.CompilerParams(
            vmem_limit_bytes=128 * 1024 * 1024),
    )(*args)


def _tc_combine(aglo, aghi, cnt16, hplo, hphi, Wl, bl, Wr, g, b):
    def body(aglo_r, aghi_r, cnt_r, hlo_r, hhi_r, wl_r, bl_r, wr_r,
             g_r, b_r, olo_r, ohi_r):
        agg = jnp.concatenate([aglo_r[...], aghi_r[...]], axis=1)
        cnt = jnp.maximum(cnt_r[...][:, :1], 1.0)
        h_prev = jnp.concatenate([hlo_r[...], hhi_r[...]], axis=1)
        z = (_mm_t(agg / cnt, wl_r[...]) + bl_r[...][None, :]
             + _mm_t(h_prev, wr_r[...]))
        res = _bn_relu(z, g_r[...][None, :], b_r[...][None, :])
        olo_r[...] = res[:, :HALF]
        ohi_r[...] = res[:, HALF:]

    f32 = jnp.float32
    out_shape = (jax.ShapeDtypeStruct((N, HALF), f32),
                 jax.ShapeDtypeStruct((N, HALF), f32))
    return _tc_call(body, out_shape, aglo, aghi, cnt16, hplo, hphi,
                    Wl, bl, Wr, g, b)


def _tc_combine_gat_prep(aglo, aghi, cnt16, hplo, hphi, Wl, bl, Wr, g, b,
                         Wg, As16, Ad16):
    def body(aglo_r, aghi_r, cnt_r, hlo_r, hhi_r, wl_r, bl_r, wr_r,
             g_r, b_r, wg_r, as_r, ad_r, xlo_r, xhi_r, aso_r, ado_r):
        agg = jnp.concatenate([aglo_r[...], aghi_r[...]], axis=1)
        cnt = jnp.maximum(cnt_r[...][:, :1], 1.0)
        h_prev = jnp.concatenate([hlo_r[...], hhi_r[...]], axis=1)
        z = (_mm_t(agg / cnt, wl_r[...]) + bl_r[...][None, :]
             + _mm_t(h_prev, wr_r[...]))
        h = _bn_relu(z, g_r[...][None, :], b_r[...][None, :])
        xl = _mm_t(h, wg_r[...])
        xlo_r[...] = xl[:, :HALF]
        xhi_r[...] = xl[:, HALF:]
        aso_r[...] = jnp.dot(xl, as_r[...],
                             preferred_element_type=jnp.float32)
        ado_r[...] = jnp.dot(xl, ad_r[...],
                             preferred_element_type=jnp.float32)

    f32 = jnp.float32
    out_shape = (jax.ShapeDtypeStruct((N, HALF), f32),
                 jax.ShapeDtypeStruct((N, HALF), f32),
                 jax.ShapeDtypeStruct((N, 16), f32),
                 jax.ShapeDtypeStruct((N, 16), f32))
    return _tc_call(body, out_shape, aglo, aghi, cnt16, hplo, hphi,
                    Wl, bl, Wr, g, b, Wg, As16, Ad16)


def _tc_final(wlo, whi, den16, S16, bg, gg, bgb, Wc1, bc1, Wc2, bc2,
              Wc3, bc3):
    def body(wlo_r, whi_r, den_r, s_r, bg_r, gg_r, bgb_r, w1_r, b1_r,
             w2_r, b2_r, w3_r, b3_r, probs_r, emb_r):
        wacc = jnp.concatenate([wlo_r[...], whi_r[...]], axis=1)
        den_b = jnp.dot(den_r[...], s_r[...],
                        preferred_element_type=jnp.float32)
        h = wacc / jnp.maximum(den_b, 1e-16) + bg_r[...][None, :]
        h = _bn_relu(h, gg_r[...][None, :], bgb_r[...][None, :])
        emb_r[...] = h
        c = jnp.maximum(_mm_t(h, w1_r[...]) + b1_r[...][None, :], 0.0)
        c = jnp.maximum(_mm_t(c, w2_r[...]) + b2_r[...][None, :], 0.0)
        logits = _mm_t(c, w3_r[...]) + b3_r[...][None, :]
        probs_r[...] = 1.0 / (1.0 + jnp.exp(-logits))

    f32 = jnp.float32
    out_shape = (jax.ShapeDtypeStruct((N, HALF), f32),
                 jax.ShapeDtypeStruct((N, D), f32))
    return _tc_call(body, out_shape, wlo, whi, den16, S16, bg, gg, bgb,
                    Wc1, bc1, Wc2, bc2, Wc3, bc3)


def kernel(x, edge_index, Wl1, bl1, Wr1, g1, b1, Wl2, bl2, Wr2, g2, b2,
           Wg, att_src, att_dst, bg, gg, bgb, Wc1, bc1, Wc2, bc2, Wc3, bc3):
    f32 = jnp.float32
    src = edge_index[0]
    dst = edge_index[1]
    x_lo = x[:, :HALF]
    x_hi = x[:, HALF:]

    lane = jnp.arange(D)
    As16 = jnp.zeros((D, 16), f32).at[lane, lane // HEAD_DIM].set(
        att_src.reshape(D))
    Ad16 = jnp.zeros((D, 16), f32).at[lane, lane // HEAD_DIM].set(
        att_dst.reshape(D))
    S16 = jnp.zeros((16, D), f32).at[lane // HEAD_DIM, lane].set(1.0)
    Wc3p = jnp.zeros((HALF, HID4), f32).at[0].set(Wc3[0])
    bc3p = jnp.zeros((HALF,), f32).at[0].set(bc3[0])

    ag1_lo, ag1_hi, cnt16 = _sc_seg_sum(x_lo, x_hi, src, dst,
                                        with_counts=True)
    cnt16 = cnt16.reshape(N, 16)
    h1_lo, h1_hi = _tc_combine(ag1_lo.reshape(N, HALF),
                               ag1_hi.reshape(N, HALF), cnt16,
                               x_lo, x_hi, Wl1, bl1, Wr1, g1, b1)

    ag2_lo, ag2_hi = _sc_seg_sum(h1_lo, h1_hi, src, dst, with_counts=False)
    xl_lo, xl_hi, as16, ad16 = _tc_combine_gat_prep(
        ag2_lo.reshape(N, HALF), ag2_hi.reshape(N, HALF), cnt16,
        h1_lo, h1_hi, Wl2, bl2, Wr2, g2, b2, Wg, As16, Ad16)

    w_lo, w_hi, den16 = _sc_gat(xl_lo, xl_hi, as16, ad16, src, dst)
    probs, emb = _tc_final(w_lo.reshape(N, HALF), w_hi.reshape(N, HALF),
                           den16.reshape(N, 16), S16, bg, gg, bgb,
                           Wc1, bc1, Wc2, bc2, Wc3p, bc3p)
    return probs[:, 0], emb


# speedup vs baseline: 1.0979x; 1.0241x over previous
import functools

import jax
import jax.numpy as jnp
from jax import lax
from jax.experimental import pallas as pl
from jax.experimental.pallas import tpu as pltpu
from jax.experimental.pallas import tpu_sc as plsc

N = 10000
E = 160000
D = 256
HEADS = 4
HEAD_DIM = 64
HALF = 128
HID4 = 64

NC = 2
NS = 16
EPT = E // NS
CH = 80
NCHUNK = EPT // CH
RPT = N // NS

_SC_MESH = dict(core_axis_name="c", subcore_axis_name="s")


def _lane_bcast(v, i):
    idx = jnp.full((16, 1), i, jnp.int32)
    dn = lax.GatherDimensionNumbers(offset_dims=(), collapsed_slice_dims=(0,),
                                    start_index_map=(0,))
    return lax.gather(v, idx, dn, (1,),
                      mode=lax.GatherScatterMode.PROMISE_IN_BOUNDS)


def _seg_sum_body(with_counts, with_gat, refs):
    if with_gat:
        (hlo, hhi, srcv, dstv, asv, adv,
         aglo, aghi, den_out,
         sidx0, sidx1, sidx2, didx0, didx1, didx2,
         rows0, rows1, rows2, asb, adb, eebuf, acc, cntacc,
         gsem0, gsem1, gsem2, ssem0, ssem1, ssem2,
         isem0, isem1, isem2) = refs
    elif with_counts:
        (hlo, hhi, srcv, dstv,
         aglo, aghi, cnt_out,
         sidx0, sidx1, sidx2, didx0, didx1, didx2,
         rows0, rows1, rows2, ones, acc, cntacc,
         gsem0, gsem1, gsem2, ssem0, ssem1, ssem2,
         isem0, isem1, isem2) = refs
    else:
        (hlo, hhi, srcv, dstv,
         aglo, aghi,
         sidx0, sidx1, sidx2, didx0, didx1, didx2,
         rows0, rows1, rows2, acc,
         gsem0, gsem1, gsem2, ssem0, ssem1, ssem2,
         isem0, isem1, isem2) = refs
    sidx = (sidx0, sidx1, sidx2)
    didx = (didx0, didx1, didx2)
    rows = (rows0, rows1, rows2)
    gsem = (gsem0, gsem1, gsem2)
    ssem = (ssem0, ssem1, ssem2)
    isem = (isem0, isem1, isem2)

    cid = lax.axis_index("c")
    sid = lax.axis_index("s")

    small = eebuf if with_gat else (ones if with_counts else None)

    @plsc.parallel_loop(0, CH, 1, unroll=4)
    def _(e):
        for j in range(8):
            rows0[e, pl.ds(j * 16, 16)] = jnp.zeros((16,), jnp.float32)
        if small is not None:
            small[e] = jnp.zeros((16,), jnp.float32)

    nfit = RPT // CH
    rem = RPT - nfit * CH
    for i in range(nfit):
        pltpu.sync_copy(rows0, acc.at[pl.ds(sid * RPT + i * CH, CH)])
        if small is not None:
            pltpu.sync_copy(small, cntacc.at[pl.ds(sid * RPT + i * CH, CH)])
    pltpu.sync_copy(rows0.at[pl.ds(0, rem)],
                    acc.at[pl.ds(sid * RPT + nfit * CH, rem)])
    if small is not None:
        pltpu.sync_copy(small.at[pl.ds(0, rem)],
                        cntacc.at[pl.ds(sid * RPT + nfit * CH, rem)])
    if with_counts:
        @plsc.parallel_loop(0, CH, 1, unroll=4)
        def _(e):
            ones[e] = jnp.ones((16,), jnp.float32)

    def idx_copies(k, b):
        base = sid * EPT + k * CH
        return ((srcv.at[pl.ds(base, CH)], sidx[b]),
                (dstv.at[pl.ds(base, CH)], didx[b]))

    def load_idx_sync(k, b):
        for s, d in idx_copies(k, b):
            pltpu.sync_copy(s, d)

    def start_idx(k, b):
        for s, d in idx_copies(k, b):
            pltpu.async_copy(s, d, isem[b])

    def wait_idx(k, b):
        for s, d in idx_copies(k, b):
            pltpu.make_async_copy(s, d, isem[b]).wait()

    def gather_copies(b):
        copies = [(hlo.at[sidx[b]], rows[b]), (hhi.at[sidx[b]], rows[b])]
        if with_gat:
            copies.append((asv.at[sidx[b]], asb))
            copies.append((adv.at[didx[b]], adb))
        return copies

    def start_gather(b):
        cps = gather_copies(b)

        @pl.when(cid == 0)
        def _():
            pltpu.async_copy(cps[0][0], cps[0][1], gsem[b])

        @pl.when(cid == 1)
        def _():
            pltpu.async_copy(cps[1][0], cps[1][1], gsem[b])

        for s, d in cps[2:]:
            pltpu.async_copy(s, d, gsem[b])

    def wait_gather(b):
        cps = gather_copies(b)
        pltpu.make_async_copy(cps[0][0], cps[0][1], gsem[b]).wait()
        for s, d in cps[2:]:
            pltpu.make_async_copy(s, d, gsem[b]).wait()

    def compute_ee(b):
        @plsc.parallel_loop(0, CH, 1, unroll=4)
        def _(e):
            ev = asb[e] + adb[e]
            ev = jnp.maximum(ev, 0.2 * ev)
            eebuf[e] = jnp.exp(ev)

    def compute_and_scatter(j, b):
        if with_gat:
            h0 = cid * 2
            rb = rows[b]

            @plsc.parallel_loop(0, CH, 1, unroll=4)
            def _(e):
                ee = eebuf[e]
                w0 = _lane_bcast(ee, h0)
                w1 = _lane_bcast(ee, h0 + 1)
                for j2 in range(8):
                    w = w0 if j2 < 4 else w1
                    rb[e, pl.ds(j2 * 16, 16)] = rb[e, pl.ds(j2 * 16, 16)] * w

            pltpu.async_copy(rb, acc.at[didx[b]], ssem[b], add=True)

            @pl.when(cid == 0)
            def _():
                pltpu.sync_copy(eebuf, cntacc.at[didx[b]], add=True)
        else:
            pltpu.async_copy(rows[b], acc.at[didx[b]], ssem[b], add=True)
            if with_counts:
                @pl.when(cid == 0)
                def _():
                    pltpu.sync_copy(ones, cntacc.at[didx[b]], add=True)

    def wait_scatter(b):
        pltpu.make_async_copy(rows[b], acc.at[didx[b]], ssem[b]).wait()

    def chunk(j, b, gather_next, prefetch_idx):
        bn = (b + 1) % 3
        bp = (b + 2) % 3
        wait_gather(b)
        if with_gat:
            compute_ee(b)
        if gather_next:
            @pl.when(j >= 1)
            def _():
                wait_idx(j + 1, bn)
            start_gather(bn)
        compute_and_scatter(j, b)

        @pl.when(j >= 1)
        def _():
            wait_scatter(bp)
        if prefetch_idx:
            start_idx(j + 2, bp)

    plsc.subcore_barrier()
    load_idx_sync(0, 0)
    load_idx_sync(1, 1)
    start_gather(0)

    def triple(j3, carry):
        chunk(j3 * 3, 0, True, True)
        chunk(j3 * 3 + 1, 1, True, True)
        chunk(j3 * 3 + 2, 2, True, True)
        return carry

    nfull = (NCHUNK - 2) // 3
    lax.fori_loop(0, nfull, triple, 0)
    chunk(NCHUNK - 2, 0, True, False)
    chunk(NCHUNK - 1, 1, False, False)
    wait_scatter((NCHUNK - 1) % 3)
    plsc.subcore_barrier()

    @pl.when(cid == 0)
    def _():
        pltpu.sync_copy(acc.at[pl.ds(sid * RPT, RPT)], aglo.at[sid])

    @pl.when(cid == 1)
    def _():
        pltpu.sync_copy(acc.at[pl.ds(sid * RPT, RPT)], aghi.at[sid])

    if with_counts or with_gat:
        small_out = cnt_out if with_counts else den_out

        @pl.when(cid == 0)
        def _():
            pltpu.sync_copy(cntacc.at[pl.ds(sid * RPT, RPT)],
                            small_out.at[sid])


def _sc_seg_sum(hlo, hhi, srcv, dstv, with_counts):
    f32 = jnp.float32
    out_type = [jax.ShapeDtypeStruct((NS, RPT, HALF), f32),
                jax.ShapeDtypeStruct((NS, RPT, HALF), f32)]
    scratch = [pltpu.VMEM((CH,), jnp.int32)] * 6 \
        + [pltpu.VMEM((CH, HALF), f32)] * 3
    args = [hlo, hhi, srcv, dstv]
    if with_counts:
        out_type.append(jax.ShapeDtypeStruct((NS, RPT, 16), f32))
        scratch.append(pltpu.VMEM((CH, 16), f32))
        scratch.append(pltpu.VMEM_SHARED((N, HALF), f32))
        scratch.append(pltpu.VMEM_SHARED((N, 16), f32))
    else:
        scratch.append(pltpu.VMEM_SHARED((N, HALF), f32))
    scratch += [pltpu.SemaphoreType.DMA] * 9

    body = functools.partial(_seg_sum_body, with_counts, False)
    fn = pl.kernel(lambda *refs: body(refs), out_type=tuple(out_type),
                   mesh=plsc.VectorSubcoreMesh(**_SC_MESH),
                   scratch_types=tuple(scratch),
                   compiler_params=pltpu.CompilerParams(
                       use_tc_tiling_on_sc=False))
    return fn(*args)


def _sc_gat(xlo, xhi, asv, adv, srcv, dstv):
    f32 = jnp.float32
    out_type = (jax.ShapeDtypeStruct((NS, RPT, HALF), f32),
                jax.ShapeDtypeStruct((NS, RPT, HALF), f32),
                jax.ShapeDtypeStruct((NS, RPT, 16), f32))
    scratch = ((pltpu.VMEM((CH,), jnp.int32),) * 6
               + (pltpu.VMEM((CH, HALF), f32),) * 3
               + (pltpu.VMEM((CH, 16), f32),) * 3
               + (pltpu.VMEM_SHARED((N, HALF), f32),
                  pltpu.VMEM_SHARED((N, 16), f32))
               + (pltpu.SemaphoreType.DMA,) * 9)
    body = functools.partial(_seg_sum_body, False, True)
    fn = pl.kernel(lambda *refs: body(refs), out_type=out_type,
                   mesh=plsc.VectorSubcoreMesh(**_SC_MESH),
                   scratch_types=scratch,
                   compiler_params=pltpu.CompilerParams(
                       use_tc_tiling_on_sc=False))
    return fn(xlo, xhi, srcv, dstv, asv, adv)


def _mm_t(a, w):
    return lax.dot_general(a, w, (((1,), (1,)), ((), ())),
                           preferred_element_type=jnp.float32)


def _bn_relu(z, g, b):
    m = jnp.mean(z, axis=0)
    v = jnp.mean((z - m) * (z - m), axis=0)
    return jnp.maximum((z - m) * lax.rsqrt(v + 1e-5) * g + b, 0.0)


def _tc_call(body, out_shape, *args):
    return pl.pallas_call(
        body, out_shape=out_shape,
        compiler_params=pltpu.CompilerParams(
            vmem_limit_bytes=128 * 1024 * 1024),
    )(*args)


def _tc_combine(aglo, aghi, cnt16, hplo, hphi, Wl, bl, Wr, g, b):
    def body(aglo_r, aghi_r, cnt_r, hlo_r, hhi_r, wl_r, bl_r, wr_r,
             g_r, b_r, olo_r, ohi_r):
        agg = jnp.concatenate([aglo_r[...], aghi_r[...]], axis=1)
        cnt = jnp.maximum(cnt_r[...][:, :1], 1.0)
        h_prev = jnp.concatenate([hlo_r[...], hhi_r[...]], axis=1)
        z = (_mm_t(agg / cnt, wl_r[...]) + bl_r[...][None, :]
             + _mm_t(h_prev, wr_r[...]))
        res = _bn_relu(z, g_r[...][None, :], b_r[...][None, :])
        olo_r[...] = res[:, :HALF]
        ohi_r[...] = res[:, HALF:]

    f32 = jnp.float32
    out_shape = (jax.ShapeDtypeStruct((N, HALF), f32),
                 jax.ShapeDtypeStruct((N, HALF), f32))
    return _tc_call(body, out_shape, aglo, aghi, cnt16, hplo, hphi,
                    Wl, bl, Wr, g, b)


def _tc_combine_gat_prep(aglo, aghi, cnt16, hplo, hphi, Wl, bl, Wr, g, b,
                         Wg, As16, Ad16):
    def body(aglo_r, aghi_r, cnt_r, hlo_r, hhi_r, wl_r, bl_r, wr_r,
             g_r, b_r, wg_r, as_r, ad_r, xlo_r, xhi_r, aso_r, ado_r):
        agg = jnp.concatenate([aglo_r[...], aghi_r[...]], axis=1)
        cnt = jnp.maximum(cnt_r[...][:, :1], 1.0)
        h_prev = jnp.concatenate([hlo_r[...], hhi_r[...]], axis=1)
        z = (_mm_t(agg / cnt, wl_r[...]) + bl_r[...][None, :]
             + _mm_t(h_prev, wr_r[...]))
        h = _bn_relu(z, g_r[...][None, :], b_r[...][None, :])
        xl = _mm_t(h, wg_r[...])
        xlo_r[...] = xl[:, :HALF]
        xhi_r[...] = xl[:, HALF:]
        aso_r[...] = jnp.dot(xl, as_r[...],
                             preferred_element_type=jnp.float32)
        ado_r[...] = jnp.dot(xl, ad_r[...],
                             preferred_element_type=jnp.float32)

    f32 = jnp.float32
    out_shape = (jax.ShapeDtypeStruct((N, HALF), f32),
                 jax.ShapeDtypeStruct((N, HALF), f32),
                 jax.ShapeDtypeStruct((N, 16), f32),
                 jax.ShapeDtypeStruct((N, 16), f32))
    return _tc_call(body, out_shape, aglo, aghi, cnt16, hplo, hphi,
                    Wl, bl, Wr, g, b, Wg, As16, Ad16)


def _tc_final(wlo, whi, den16, S16, bg, gg, bgb, Wc1, bc1, Wc2, bc2,
              Wc3, bc3):
    def body(wlo_r, whi_r, den_r, s_r, bg_r, gg_r, bgb_r, w1_r, b1_r,
             w2_r, b2_r, w3_r, b3_r, probs_r, emb_r):
        wacc = jnp.concatenate([wlo_r[...], whi_r[...]], axis=1)
        den_b = jnp.dot(den_r[...], s_r[...],
                        preferred_element_type=jnp.float32)
        h = wacc / jnp.maximum(den_b, 1e-16) + bg_r[...][None, :]
        h = _bn_relu(h, gg_r[...][None, :], bgb_r[...][None, :])
        emb_r[...] = h
        c = jnp.maximum(_mm_t(h, w1_r[...]) + b1_r[...][None, :], 0.0)
        c = jnp.maximum(_mm_t(c, w2_r[...]) + b2_r[...][None, :], 0.0)
        logits = _mm_t(c, w3_r[...]) + b3_r[...][None, :]
        probs_r[...] = 1.0 / (1.0 + jnp.exp(-logits))

    f32 = jnp.float32
    out_shape = (jax.ShapeDtypeStruct((N, HALF), f32),
                 jax.ShapeDtypeStruct((N, D), f32))
    return _tc_call(body, out_shape, wlo, whi, den16, S16, bg, gg, bgb,
                    Wc1, bc1, Wc2, bc2, Wc3, bc3)


def kernel(x, edge_index, Wl1, bl1, Wr1, g1, b1, Wl2, bl2, Wr2, g2, b2,
           Wg, att_src, att_dst, bg, gg, bgb, Wc1, bc1, Wc2, bc2, Wc3, bc3):
    f32 = jnp.float32
    src = edge_index[0]
    dst = edge_index[1]
    x_lo = x[:, :HALF]
    x_hi = x[:, HALF:]

    lane = jnp.arange(D)
    As16 = jnp.zeros((D, 16), f32).at[lane, lane // HEAD_DIM].set(
        att_src.reshape(D))
    Ad16 = jnp.zeros((D, 16), f32).at[lane, lane // HEAD_DIM].set(
        att_dst.reshape(D))
    S16 = jnp.zeros((16, D), f32).at[lane // HEAD_DIM, lane].set(1.0)
    Wc3p = jnp.zeros((HALF, HID4), f32).at[0].set(Wc3[0])
    bc3p = jnp.zeros((HALF,), f32).at[0].set(bc3[0])

    ag1_lo, ag1_hi, cnt16 = _sc_seg_sum(x_lo, x_hi, src, dst,
                                        with_counts=True)
    cnt16 = cnt16.reshape(N, 16)
    h1_lo, h1_hi = _tc_combine(ag1_lo.reshape(N, HALF),
                               ag1_hi.reshape(N, HALF), cnt16,
                               x_lo, x_hi, Wl1, bl1, Wr1, g1, b1)

    ag2_lo, ag2_hi = _sc_seg_sum(h1_lo, h1_hi, src, dst, with_counts=False)
    xl_lo, xl_hi, as16, ad16 = _tc_combine_gat_prep(
        ag2_lo.reshape(N, HALF), ag2_hi.reshape(N, HALF), cnt16,
        h1_lo, h1_hi, Wl2, bl2, Wr2, g2, b2, Wg, As16, Ad16)

    w_lo, w_hi, den16 = _sc_gat(xl_lo, xl_hi, as16, ad16, src, dst)
    probs, emb = _tc_final(w_lo.reshape(N, HALF), w_hi.reshape(N, HALF),
                           den16.reshape(N, 16), S16, bg, gg, bgb,
                           Wc1, bc1, Wc2, bc2, Wc3p, bc3p)
    return probs[:, 0], emb
